# baseline (speedup 1.0000x reference)
import functools

import jax
import jax.numpy as jnp
from jax import lax
from jax.experimental import pallas as pl
from jax.experimental.pallas import tpu as pltpu
from jax.experimental.pallas import tpu_sc as plsc

N = 10000
E = 320000
D = 128
H = 128
C = 16
T = 24

NC = 2
NS = 16
NW = NC * NS
CH1, CHUNK1 = 90, 120
CH2, CHUNK2 = 84, 128
NA1 = 10240
NA2 = 10016
NT = 10016
W1 = 144
W2 = 16
BLK = 1000


def _sc_pass(w):
    wide = w > 64
    if wide:
        ch, chunk, n_acc = CH1, CHUNK1, NA1
    else:
        ch, chunk, n_acc = CH2, CHUNK2, NA2
        nbr, la = 4, 2
    rpt = n_acc // NS
    mesh = plsc.VectorSubcoreMesh(core_axis_name="c", subcore_axis_name="s")

    if wide:
        scratch = [
            pltpu.VMEM((ch, chunk), jnp.int32),
            pltpu.VMEM((ch, chunk), jnp.int32),
            pltpu.VMEM((chunk, w), jnp.float32),
            pltpu.VMEM_SHARED((n_acc, w), jnp.float32),
            pltpu.SemaphoreType.DMA,
        ]
    else:
        scratch = [
            pltpu.VMEM((ch, chunk), jnp.int32),
            pltpu.VMEM((ch, chunk), jnp.int32),
            pltpu.VMEM((nbr, chunk, w), jnp.float32),
            pltpu.VMEM_SHARED((n_acc, w), jnp.float32),
            pltpu.SemaphoreType.DMA((nbr,)),
            pltpu.SemaphoreType.DMA((nbr,)),
        ]

    @functools.partial(
        pl.kernel,
        mesh=mesh,
        compiler_params=pltpu.CompilerParams(use_tc_tiling_on_sc=False),
        out_type=jax.ShapeDtypeStruct((NC, n_acc, w), jnp.float32),
        scratch_types=scratch,
    )
    def pass_kernel(tbl_hbm, src_hbm, dst_hbm, zeros_hbm, out_hbm,
                    src_v, dst_v, rows_v, acc, *sems):
        c = lax.axis_index("c")
        s = lax.axis_index("s")
        wid = c * NS + s
        pltpu.sync_copy(zeros_hbm, acc.at[pl.ds(s * rpt, rpt)])
        pltpu.sync_copy(src_hbm.at[wid], src_v)
        pltpu.sync_copy(dst_hbm.at[wid], dst_v)

        if wide:
            (sem,) = sems
            plsc.subcore_barrier()

            def body(j, carry):
                pltpu.async_copy(tbl_hbm.at[src_v.at[j]], rows_v, sem).wait()
                pltpu.sync_copy(rows_v, acc.at[dst_v.at[j]], add=True)
                return carry

            lax.fori_loop(0, ch, body, 0)
        else:
            sem_g, sem_s = sems

            def gather_start(k, br):
                pltpu.async_copy(tbl_hbm.at[src_v.at[k]], rows_v.at[br],
                                 sem_g.at[br])

            def gather_wait(k, br):
                pltpu.make_async_copy(tbl_hbm.at[src_v.at[k]],
                                      rows_v.at[br], sem_g.at[br]).wait()

            def scatter_start(k, br):
                pltpu.async_copy(rows_v.at[br], acc.at[dst_v.at[k]],
                                 sem_s.at[br], add=True)

            def scatter_wait(br):
                pltpu.make_async_copy(rows_v.at[br], acc.at[dst_v.at[0]],
                                      sem_s.at[br]).wait()

            for k in range(la):
                gather_start(k, k % nbr)
            plsc.subcore_barrier()

            def body(g, carry):
                for u in range(nbr):
                    k = g * nbr + u
                    gather_wait(k, u)
                    scatter_start(k, u)
                    k2 = k + la
                    br2 = (u + la) % nbr

                    @pl.when(k2 < ch)
                    def _():
                        @pl.when(k2 >= nbr)
                        def _():
                            scatter_wait(br2)
                        gather_start(k2, br2)
                return carry

            lax.fori_loop(0, ch // nbr, body, 0)
            for b in range(nbr):
                scatter_wait(b)

        plsc.subcore_barrier()
        pltpu.sync_copy(acc.at[pl.ds(s * rpt, rpt)],
                        out_hbm.at[c].at[pl.ds(s * rpt, rpt)])

    return pass_kernel


_sc_pass = functools.lru_cache(maxsize=None)(_sc_pass)


def _k1_body(feats_ref, ct_ref, emb_ref, w1s_ref, w1n_ref, b1_ref,
             tbl_ref, xw1s_ref):
    ct = ct_ref[...]
    tt = lax.broadcasted_iota(jnp.int32, (BLK, T), 1)
    onehot = (ct == tt).astype(jnp.float32)
    w = jnp.dot(onehot, emb_ref[...], preferred_element_type=jnp.float32)
    x = feats_ref[...] * jax.nn.sigmoid(w)
    xw1n = jnp.dot(x, w1n_ref[...], preferred_element_type=jnp.float32)
    tbl_ref[:, pl.ds(0, D)] = xw1n
    tbl_ref[:, pl.ds(D, W1 - D)] = jnp.ones((BLK, W1 - D), jnp.float32)
    xw1s_ref[...] = (
        jnp.dot(x, w1s_ref[...], preferred_element_type=jnp.float32)
        + b1_ref[...]
    )


def _k2_body(a0_ref, a1_ref, xw1s_ref, w2s_ref, w2n_ref, b2_ref,
             tbl2_ref, hw2s_ref, inv_ref):
    a = a0_ref[...] + a1_ref[...]
    cnt = a[:, D:D + 1]
    inv = 1.0 / jnp.maximum(cnt, 1.0)
    h = jax.nn.relu(xw1s_ref[...] + a[:, :D] * inv)
    tbl2_ref[...] = jnp.dot(h, w2n_ref[...], preferred_element_type=jnp.float32)
    hw2s_ref[...] = (
        jnp.dot(h, w2s_ref[...], preferred_element_type=jnp.float32)
        + b2_ref[...]
    )
    inv_ref[...] = jnp.broadcast_to(inv, (BLK, C))


def _k3_body(s0_ref, s1_ref, hw2s_ref, inv_ref, out_ref):
    out_ref[...] = hw2s_ref[...] + (s0_ref[...] + s1_ref[...]) * inv_ref[...]


def _full(shape):
    return pl.BlockSpec(shape, lambda i: (0, 0))


def _rows(width):
    return pl.BlockSpec((BLK, width), lambda i: (i, 0))


def kernel(feats, cell_type_ids, edge_index, emb, W1_self, W1_neigh, b1,
           W2_self, W2_neigh, b2):
    grid = (N // BLK,)
    ct2d = cell_type_ids.astype(jnp.int32).reshape(N, 1)

    tbl1, xw1s = pl.pallas_call(
        _k1_body,
        grid=grid,
        in_specs=[
            _rows(D), pl.BlockSpec((BLK, 1), lambda i: (i, 0)),
            _full((T, D)), _full((D, H)), _full((D, H)), _full((1, H)),
        ],
        out_specs=[_rows(W1), _rows(H)],
        out_shape=[
            jax.ShapeDtypeStruct((N, W1), jnp.float32),
            jax.ShapeDtypeStruct((N, H), jnp.float32),
        ],
    )(feats, ct2d, emb, W1_self, W1_neigh, b1.reshape(1, H))

    esc = E // NC
    main = 15 * (esc // 15)

    def edge_layout(flat, padval, ch, chunk):
        cap = ch * chunk
        per_sc = []
        for cidx in range(NC):
            e = flat[cidx * esc:(cidx + 1) * esc]
            body15 = e[:main].reshape(15, esc // 15)
            body15 = jnp.concatenate(
                [body15,
                 jnp.full((15, cap - esc // 15), padval, jnp.int32)], axis=1)
            tail = jnp.concatenate(
                [e[main:], jnp.full((cap - (esc - main),), padval,
                                    jnp.int32)])
            per_sc.append(jnp.concatenate([body15, tail[None]], axis=0))
        return jnp.concatenate(per_sc, axis=0).reshape(NW, ch, chunk)

    src = edge_index[0].astype(jnp.int32)
    dst = edge_index[1].astype(jnp.int32)
    srcp1 = edge_layout(src, N, CH1, CHUNK1)
    dstp1 = edge_layout(dst, 0, CH1, CHUNK1)
    srcp2 = edge_layout(src, N, CH2, CHUNK2)
    dstp2 = edge_layout(dst, 0, CH2, CHUNK2)

    tbl1_full = jnp.concatenate(
        [tbl1, jnp.zeros((NT - N, W1), jnp.float32)], axis=0)
    acc1 = _sc_pass(W1)(tbl1_full, srcp1, dstp1,
                        jnp.zeros((NA1 // NS, W1), jnp.float32))

    tbl2, hw2s, inv16 = pl.pallas_call(
        _k2_body,
        grid=grid,
        in_specs=[
            _rows(W1), _rows(W1), _rows(H),
            _full((H, C)), _full((H, C)), _full((1, C)),
        ],
        out_specs=[_rows(C), _rows(C), _rows(C)],
        out_shape=[
            jax.ShapeDtypeStruct((N, C), jnp.float32),
            jax.ShapeDtypeStruct((N, C), jnp.float32),
            jax.ShapeDtypeStruct((N, C), jnp.float32),
        ],
    )(acc1[0, :N], acc1[1, :N], xw1s, W2_self, W2_neigh, b2.reshape(1, C))

    tbl2_full = jnp.concatenate(
        [tbl2, jnp.zeros((NT - N, W2), jnp.float32)], axis=0)
    acc2 = _sc_pass(W2)(tbl2_full, srcp2, dstp2,
                        jnp.zeros((NA2 // NS, W2), jnp.float32))

    out = pl.pallas_call(
        _k3_body,
        grid=grid,
        in_specs=[_rows(C), _rows(C), _rows(C), _rows(C)],
        out_specs=_rows(C),
        out_shape=jax.ShapeDtypeStruct((N, C), jnp.float32),
    )(acc2[0, :N], acc2[1, :N], hw2s, inv16)

    return out


# speedup vs baseline: 3.4300x; 3.4300x over previous
import functools

import jax
import jax.numpy as jnp
from jax import lax
from jax.experimental import pallas as pl
from jax.experimental.pallas import tpu as pltpu
from jax.experimental.pallas import tpu_sc as plsc

N = 10000
E = 320000
D = 128
H = 128
C = 16
T = 24

NC = 2
NS = 16
NW = NC * NS
CH1, CHUNK1 = 79, 128
CH2, CHUNK2 = 80, 128
NA1 = 10240
NA2 = 10016
NT = N + 4096
W1 = 144
W2 = 16
BLK = 1000


def _sc_pass(w):
    wide = w > 64
    if wide:
        ch, chunk, n_acc = CH1, CHUNK1, NA1
    else:
        ch, chunk, n_acc = CH2, CHUNK2, NA2
        nbr, la = 4, 2
    rpt = n_acc // NS
    mesh = plsc.VectorSubcoreMesh(core_axis_name="c", subcore_axis_name="s")

    if wide:
        scratch = [
            pltpu.VMEM((ch, chunk), jnp.int32),
            pltpu.VMEM((ch, chunk), jnp.int32),
            pltpu.VMEM((chunk, w), jnp.float32),
            pltpu.VMEM_SHARED((n_acc, w), jnp.float32),
            pltpu.SemaphoreType.DMA,
        ]
    else:
        scratch = [
            pltpu.VMEM((ch, chunk), jnp.int32),
            pltpu.VMEM((ch, chunk), jnp.int32),
            pltpu.VMEM((nbr, chunk, w), jnp.float32),
            pltpu.VMEM_SHARED((n_acc, w), jnp.float32),
            pltpu.SemaphoreType.DMA((nbr,)),
            pltpu.SemaphoreType.DMA((nbr,)),
        ]

    @functools.partial(
        pl.kernel,
        mesh=mesh,
        compiler_params=pltpu.CompilerParams(use_tc_tiling_on_sc=False),
        out_type=jax.ShapeDtypeStruct((NC, n_acc, w), jnp.float32),
        scratch_types=scratch,
    )
    def pass_kernel(tbl_hbm, src_hbm, dst_hbm, zeros_hbm, out_hbm,
                    src_v, dst_v, rows_v, acc, *sems):
        c = lax.axis_index("c")
        s = lax.axis_index("s")
        wid = c * NS + s
        pltpu.sync_copy(zeros_hbm, acc.at[pl.ds(s * rpt, rpt)])
        pltpu.sync_copy(src_hbm.at[wid], src_v)
        pltpu.sync_copy(dst_hbm.at[wid], dst_v)

        if wide:
            (sem,) = sems
            plsc.subcore_barrier()

            def body(j, carry):
                pltpu.async_copy(tbl_hbm.at[src_v.at[j]], rows_v, sem).wait()
                pltpu.sync_copy(rows_v, acc.at[dst_v.at[j]], add=True)
                return carry

            lax.fori_loop(0, ch, body, 0)
        else:
            sem_g, sem_s = sems

            def gather_start(k, br):
                pltpu.async_copy(tbl_hbm.at[src_v.at[k]], rows_v.at[br],
                                 sem_g.at[br])

            def gather_wait(k, br):
                pltpu.make_async_copy(tbl_hbm.at[src_v.at[k]],
                                      rows_v.at[br], sem_g.at[br]).wait()

            def scatter_start(k, br):
                pltpu.async_copy(rows_v.at[br], acc.at[dst_v.at[k]],
                                 sem_s.at[br], add=True)

            def scatter_wait(br):
                pltpu.make_async_copy(rows_v.at[br], acc.at[dst_v.at[0]],
                                      sem_s.at[br]).wait()

            for k in range(la):
                gather_start(k, k % nbr)
            plsc.subcore_barrier()

            def body(g, carry):
                for u in range(nbr):
                    k = g * nbr + u
                    gather_wait(k, u)
                    scatter_start(k, u)
                    k2 = k + la
                    br2 = (u + la) % nbr

                    @pl.when(k2 < ch)
                    def _():
                        @pl.when(k2 >= nbr)
                        def _():
                            scatter_wait(br2)
                        gather_start(k2, br2)
                return carry

            lax.fori_loop(0, ch // nbr, body, 0)
            for b in range(nbr):
                scatter_wait(b)

        plsc.subcore_barrier()
        pltpu.sync_copy(acc.at[pl.ds(s * rpt, rpt)],
                        out_hbm.at[c].at[pl.ds(s * rpt, rpt)])

    return pass_kernel


_sc_pass = functools.lru_cache(maxsize=None)(_sc_pass)


def _k1_body(feats_ref, ct_ref, emb_ref, w1s_ref, w1n_ref, b1_ref,
             tbl_ref, xw1s_ref):
    ct = ct_ref[...]
    tt = lax.broadcasted_iota(jnp.int32, (BLK, T), 1)
    onehot = (ct == tt).astype(jnp.float32)
    w = jnp.dot(onehot, emb_ref[...], preferred_element_type=jnp.float32)
    x = feats_ref[...] * jax.nn.sigmoid(w)
    xw1n = jnp.dot(x, w1n_ref[...], preferred_element_type=jnp.float32)
    tbl_ref[:, pl.ds(0, D)] = xw1n
    tbl_ref[:, pl.ds(D, W1 - D)] = jnp.ones((BLK, W1 - D), jnp.float32)
    xw1s_ref[...] = (
        jnp.dot(x, w1s_ref[...], preferred_element_type=jnp.float32)
        + b1_ref[...]
    )


def _k2_body(a0_ref, a1_ref, xw1s_ref, w2s_ref, w2n_ref, b2_ref,
             tbl2_ref, hw2s_ref, inv_ref):
    a = a0_ref[...] + a1_ref[...]
    cnt = a[:, D:D + 1]
    inv = 1.0 / jnp.maximum(cnt, 1.0)
    h = jax.nn.relu(xw1s_ref[...] + a[:, :D] * inv)
    tbl2_ref[...] = jnp.dot(h, w2n_ref[...], preferred_element_type=jnp.float32)
    hw2s_ref[...] = (
        jnp.dot(h, w2s_ref[...], preferred_element_type=jnp.float32)
        + b2_ref[...]
    )
    inv_ref[...] = jnp.broadcast_to(inv, (BLK, C))


def _k3_body(s0_ref, s1_ref, hw2s_ref, inv_ref, out_ref):
    out_ref[...] = hw2s_ref[...] + (s0_ref[...] + s1_ref[...]) * inv_ref[...]


def _full(shape):
    return pl.BlockSpec(shape, lambda i: (0, 0))


def _rows(width):
    return pl.BlockSpec((BLK, width), lambda i: (i, 0))


def kernel(feats, cell_type_ids, edge_index, emb, W1_self, W1_neigh, b1,
           W2_self, W2_neigh, b2):
    grid = (N // BLK,)
    ct2d = cell_type_ids.astype(jnp.int32).reshape(N, 1)

    tbl1, xw1s = pl.pallas_call(
        _k1_body,
        grid=grid,
        in_specs=[
            _rows(D), pl.BlockSpec((BLK, 1), lambda i: (i, 0)),
            _full((T, D)), _full((D, H)), _full((D, H)), _full((1, H)),
        ],
        out_specs=[_rows(W1), _rows(H)],
        out_shape=[
            jax.ShapeDtypeStruct((N, W1), jnp.float32),
            jax.ShapeDtypeStruct((N, H), jnp.float32),
        ],
    )(feats, ct2d, emb, W1_self, W1_neigh, b1.reshape(1, H))

    src = edge_index[0].astype(jnp.int32)
    dst = edge_index[1].astype(jnp.int32)

    def edge_layout(ch, chunk):
        pad = NW * ch * chunk - E
        pad_src = N + (jnp.arange(pad, dtype=jnp.int32) % (NT - N))
        pad_dst = jnp.arange(pad, dtype=jnp.int32) % N
        srcp = jnp.concatenate([src, pad_src]).reshape(NW, ch, chunk)
        dstp = jnp.concatenate([dst, pad_dst]).reshape(NW, ch, chunk)
        return srcp, dstp

    srcp1, dstp1 = edge_layout(CH1, CHUNK1)
    srcp2, dstp2 = edge_layout(CH2, CHUNK2)

    tbl1_full = jnp.concatenate(
        [tbl1, jnp.zeros((NT - N, W1), jnp.float32)], axis=0)
    acc1 = _sc_pass(W1)(tbl1_full, srcp1, dstp1,
                        jnp.zeros((NA1 // NS, W1), jnp.float32))

    tbl2, hw2s, inv16 = pl.pallas_call(
        _k2_body,
        grid=grid,
        in_specs=[
            _rows(W1), _rows(W1), _rows(H),
            _full((H, C)), _full((H, C)), _full((1, C)),
        ],
        out_specs=[_rows(C), _rows(C), _rows(C)],
        out_shape=[
            jax.ShapeDtypeStruct((N, C), jnp.float32),
            jax.ShapeDtypeStruct((N, C), jnp.float32),
            jax.ShapeDtypeStruct((N, C), jnp.float32),
        ],
    )(acc1[0, :N], acc1[1, :N], xw1s, W2_self, W2_neigh, b2.reshape(1, C))

    tbl2_full = jnp.concatenate(
        [tbl2, jnp.zeros((NT - N, W2), jnp.float32)], axis=0)
    acc2 = _sc_pass(W2)(tbl2_full, srcp2, dstp2,
                        jnp.zeros((NA2 // NS, W2), jnp.float32))

    out = pl.pallas_call(
        _k3_body,
        grid=grid,
        in_specs=[_rows(C), _rows(C), _rows(C), _rows(C)],
        out_specs=_rows(C),
        out_shape=jax.ShapeDtypeStruct((N, C), jnp.float32),
    )(acc2[0, :N], acc2[1, :N], hw2s, inv16)

    return out
